# baseline (speedup 1.0000x reference)
import functools

import jax
import jax.numpy as jnp
from jax import lax
from jax.experimental import pallas as pl
from jax.experimental.pallas import tpu as pltpu
from jax.experimental.pallas import tpu_sc as plsc

EMBED = 256
HEADS = 8
HD = EMBED // HEADS
KC = 128
KF = 512
B = 8

_PREC = lax.Precision.DEFAULT


def _dot(a, b):
    return lax.dot_general(a, b, (((1,), (0,)), ((), ())),
                           preferred_element_type=jnp.float32, precision=_PREC)


def _dotT(a, b):
    return lax.dot_general(a, b, (((1,), (1,)), ((), ())),
                           preferred_element_type=jnp.float32, precision=_PREC)


def _dotL(a, b):
    return lax.dot_general(a, b, (((0,), (1,)), ((), ())),
                           preferred_element_type=jnp.float32, precision=_PREC)


def _ln_rows(x, w, b):
    m = jnp.mean(x, axis=-1, keepdims=True)
    v = jnp.mean((x - m) ** 2, axis=-1, keepdims=True)
    return (x - m) / jnp.sqrt(v + 1e-5) * w + b


def _silu(x):
    return x * jax.nn.sigmoid(x)


def _tok_body(x_ref, win_ref, nw_ref, nb_ref, w1_ref, b1_ref, w2_ref, b2_ref,
              t_ref, s_ref):
    x = x_ref[0]
    t = _dotL(x, win_ref[...])
    t_ref[0] = t
    m = jnp.sum(t, axis=-1, keepdims=True) * (1.0 / EMBED)
    v = jnp.sum(t * t, axis=-1, keepdims=True) * (1.0 / EMBED) - m * m
    inv = 1.0 / jnp.sqrt(v + 1e-5)
    nt = (t - m) * inv * nw_ref[...] + nb_ref[...]
    h = _silu(_dotT(nt, w1_ref[...]) + b1_ref[...])
    hb = h.astype(jnp.bfloat16).astype(jnp.float32)
    wb = w2_ref[...].astype(jnp.bfloat16).astype(jnp.float32)
    s = jnp.sum(hb * wb, axis=-1) + b2_ref[0, 0]
    s_ref[0, 0, 0] = s


def _tokenize(x, win, nw, nb, w1, b1, w2, b2, chunk):
    Bx, C, S = x.shape
    grid = (Bx, S // chunk)
    return pl.pallas_call(
        _tok_body,
        grid=grid,
        in_specs=[
            pl.BlockSpec((1, C, chunk), lambda b, s: (b, 0, s)),
            pl.BlockSpec((EMBED, C), lambda b, s: (0, 0)),
            pl.BlockSpec((1, EMBED), lambda b, s: (0, 0)),
            pl.BlockSpec((1, EMBED), lambda b, s: (0, 0)),
            pl.BlockSpec((64, EMBED), lambda b, s: (0, 0)),
            pl.BlockSpec((1, 64), lambda b, s: (0, 0)),
            pl.BlockSpec((1, 64), lambda b, s: (0, 0)),
            pl.BlockSpec((1, 1), lambda b, s: (0, 0), memory_space=pltpu.SMEM),
        ],
        out_specs=[
            pl.BlockSpec((1, chunk, EMBED), lambda b, s: (b, s, 0)),
            pl.BlockSpec((1, 1, 1, chunk), lambda b, s: (b, s, 0, 0)),
        ],
        out_shape=[
            jax.ShapeDtypeStruct((Bx, S, EMBED), jnp.float32),
            jax.ShapeDtypeStruct((Bx, S // chunk, 1, chunk), jnp.float32),
        ],
    )(x, win, nw, nb, w1, b1, w2, b2)


def _monotone_u32(s):
    raw = lax.bitcast_convert_type(s, jnp.uint32)
    top = jnp.uint32(0x80000000)
    return jnp.where(raw >= top, ~raw, raw | top)


def _cumsum_lanes(x):
    Bx, S = x.shape
    d = 1
    while d < S:
        sh = jnp.concatenate([jnp.zeros((Bx, d), x.dtype), x[:, :-d]], axis=1)
        x = x + sh
        d *= 2
    return x


def _kth_threshold(u, k):
    Bx = u.shape[0]
    T0 = jnp.zeros((Bx, 1), jnp.uint32)

    def body(i, T):
        bit = lax.shift_left(jnp.uint32(1), jnp.uint32(31) - i.astype(jnp.uint32))
        Ttry = T | bit
        c = jnp.sum((u >= Ttry).astype(jnp.int32), axis=1, keepdims=True)
        return jnp.where(c >= k, Ttry, T)

    return lax.fori_loop(0, 32, body, T0)


def _select_mask(u, k):
    T = _kth_threshold(u, k)
    gt = u > T
    eq = u == T
    n_gt = jnp.sum(gt.astype(jnp.int32), axis=1, keepdims=True)
    need = k - n_gt
    excl = _cumsum_lanes(eq.astype(jnp.int32)) - eq.astype(jnp.int32)
    return gt | (eq & (excl < need))


def _compact(sel, k, s_chunk):
    Bx, S = sel.shape
    c = _cumsum_lanes(sel.astype(jnp.float32))
    jg = lax.broadcasted_iota(jnp.int32, (1, 1, k), 2).astype(jnp.float32)
    acc = jnp.zeros((Bx, k), jnp.float32)
    for s0 in range(0, S, s_chunk):
        cc = c[:, s0:s0 + s_chunk, None]
        acc = acc + jnp.sum((cc <= jg).astype(jnp.float32), axis=1)
    return acc.astype(jnp.int32)


def _select_body(ks, s_ref, *out_refs):
    s = s_ref[...]
    u = _monotone_u32(s)
    Bx, S = s.shape
    for k, ref in zip(ks, out_refs):
        sel = _select_mask(u, k)
        loc = _compact(sel, k, min(S, 512))
        ref[...] = loc + S * lax.broadcasted_iota(jnp.int32, (Bx, k), 0)


def _topk_idx(scores, ks):
    Bx, S = scores.shape
    return pl.pallas_call(
        functools.partial(_select_body, ks),
        in_specs=[pl.BlockSpec((Bx, S), lambda: (0, 0))],
        out_specs=[pl.BlockSpec((Bx, k), lambda: (0, 0)) for k in ks],
        out_shape=[jax.ShapeDtypeStruct((Bx, k), jnp.int32) for k in ks],
    )(scores)


_SC_MESH = plsc.VectorSubcoreMesh(core_axis_name="c", subcore_axis_name="s")


def _gather_all(t0, t1, t2, ic0, ic1, ic2, fi0, fi1):
    CPER = B * KC // 32
    FPER = B * KF // 32

    @functools.partial(
        pl.kernel,
        out_type=(jax.ShapeDtypeStruct((B * KC, EMBED), jnp.float32),
                  jax.ShapeDtypeStruct((B * KC, EMBED), jnp.float32),
                  jax.ShapeDtypeStruct((B * KC, EMBED), jnp.float32),
                  jax.ShapeDtypeStruct((B * KF, EMBED), jnp.float32),
                  jax.ShapeDtypeStruct((B * KF, EMBED), jnp.float32)),
        mesh=_SC_MESH,
        scratch_types=[
            pltpu.VMEM((CPER,), jnp.int32),
            pltpu.VMEM((CPER, EMBED), jnp.float32),
            pltpu.VMEM((FPER,), jnp.int32),
            pltpu.VMEM((FPER, EMBED), jnp.float32),
            pltpu.SemaphoreType.DMA,
        ],
    )
    def k(t0_h, t1_h, t2_h, i0_h, i1_h, i2_h, f0_h, f1_h,
          g0_h, g1_h, g2_h, o0_h, o1_h, ci_v, cr_v, fi_v, fr_v, sem):
        w = lax.axis_index("s") * 2 + lax.axis_index("c")
        cbase = w * CPER
        fbase = w * FPER
        for th, ih, gh in ((t0_h, i0_h, g0_h), (t1_h, i1_h, g1_h),
                           (t2_h, i2_h, g2_h)):
            pltpu.sync_copy(ih.at[pl.ds(cbase, CPER)], ci_v)
            pltpu.async_copy(th.at[ci_v], cr_v, sem).wait()
            pltpu.sync_copy(cr_v, gh.at[pl.ds(cbase, CPER)])
        for th, fh, oh in ((t0_h, f0_h, o0_h), (t1_h, f1_h, o1_h)):
            pltpu.sync_copy(fh.at[pl.ds(fbase, FPER)], fi_v)
            pltpu.async_copy(th.at[fi_v], fr_v, sem).wait()
            pltpu.sync_copy(fr_v, oh.at[pl.ds(fbase, FPER)])

    return k(t0, t1, t2, ic0, ic1, ic2, fi0, fi1)


def _scatter_rows(t0, fi0, rows0, S0, t1, fi1, rows1, S1):
    SCAT = 4 * KF // 16

    @functools.partial(
        pl.kernel,
        out_type=(jax.ShapeDtypeStruct((B * S0, EMBED), jnp.float32),
                  jax.ShapeDtypeStruct((B * S1, EMBED), jnp.float32)),
        mesh=_SC_MESH,
        scratch_types=[
            pltpu.VMEM((SCAT,), jnp.int32),
            pltpu.VMEM((SCAT, EMBED), jnp.float32),
            pltpu.SemaphoreType.DMA,
        ],
    )
    def k(t0_h, f0_h, r0_h, t1_h, f1_h, r1_h, o0_h, o1_h,
          idx_v, rbuf_v, sem):
        cid = lax.axis_index("c")
        sid = lax.axis_index("s")
        for S, t_h, o_h in ((S0, t0_h, o0_h), (S1, t1_h, o1_h)):
            cp = S // 4
            cbase = cid * (4 * S) + sid * cp
            pltpu.sync_copy(t_h.at[pl.ds(cbase, cp)],
                            o_h.at[pl.ds(cbase, cp)])
        plsc.subcore_barrier()
        sbase = cid * (4 * KF) + sid * SCAT
        for f_h, r_h, o_h in ((f0_h, r0_h, o0_h), (f1_h, r1_h, o1_h)):
            pltpu.sync_copy(f_h.at[pl.ds(sbase, SCAT)], idx_v)
            pltpu.sync_copy(r_h.at[pl.ds(sbase, SCAT)], rbuf_v)
            pltpu.async_copy(rbuf_v, o_h.at[idx_v], sem).wait()

    return k(t0, fi0, rows0, t1, fi1, rows1)


def _attn(q, k, v):
    o = []
    sc = HD ** (-0.5)
    for h in range(HEADS):
        qh = q[:, h * HD:(h + 1) * HD]
        kh = k[:, h * HD:(h + 1) * HD]
        vh = v[:, h * HD:(h + 1) * HD]
        e = jnp.exp(_dotT(qh, kh) * sc)
        r = 1.0 / jnp.sum(e, axis=-1, keepdims=True)
        o.append(_dot(e, vh) * r)
    return jnp.concatenate(o, axis=-1)


def _coarse_body(cc_ref, wq, bq, wk, bk, wv, bv, wp, bp, nw, nb,
                 fw1, fb1, fw2, fb2, cu_ref):
    X = cc_ref[0]
    q = _dotT(X, wq[...]) + bq[...]
    k = _dotT(X, wk[...]) + bk[...]
    v = _dotT(X, wv[...]) + bv[...]
    o = _attn(q, k, v)
    cu0 = _dotT(o, wp[...]) + bp[...]
    f = _silu(_dotT(_ln_rows(cu0, nw[...], nb[...]), fw1[...]) + fb1[...])
    cu_ref[0] = _dotT(f, fw2[...]) + fb2[...] + cu0


def _coarse_attn(cc, ws):
    N = cc.shape[1]
    specs = [pl.BlockSpec((1, N, EMBED), lambda b: (b, 0, 0))]
    for w in ws:
        sh = w.shape
        specs.append(pl.BlockSpec(sh, lambda b: (0,) * len(sh)))
    return pl.pallas_call(
        _coarse_body,
        grid=(cc.shape[0],),
        in_specs=specs,
        out_specs=pl.BlockSpec((1, N, EMBED), lambda b: (b, 0, 0)),
        out_shape=jax.ShapeDtypeStruct(cc.shape, jnp.float32),
    )(cc, *ws)


def _cross_body(fs_ref, cu_ref, wq, bq, wk, bk, wv, bv, wp, bp, nw, nb,
                fw1, fb1, fw2, fb2, out_ref):
    F = fs_ref[0]
    CU = cu_ref[0]
    q = _dotT(F, wq[...]) + bq[...]
    k = _dotT(CU, wk[...]) + bk[...]
    v = _dotT(CU, wv[...]) + bv[...]
    o = _attn(q, k, v)
    up0 = _dotT(o, wp[...]) + bp[...]
    f = _silu(_dotT(_ln_rows(up0, nw[...], nb[...]), fw1[...]) + fb1[...])
    up = _dotT(f, fw2[...]) + fb2[...] + up0
    out_ref[0] = F + up


def _cross_attn(fs, cu, ws):
    N = fs.shape[1]
    M = cu.shape[1]
    specs = [pl.BlockSpec((1, N, EMBED), lambda b: (b, 0, 0)),
             pl.BlockSpec((1, M, EMBED), lambda b: (b, 0, 0))]
    for w in ws:
        sh = w.shape
        specs.append(pl.BlockSpec(sh, lambda b: (0,) * len(sh)))
    return pl.pallas_call(
        _cross_body,
        grid=(fs.shape[0],),
        in_specs=specs,
        out_specs=pl.BlockSpec((1, N, EMBED), lambda b: (b, 0, 0)),
        out_shape=jax.ShapeDtypeStruct(fs.shape, jnp.float32),
    )(fs, cu, *ws)


def _proj_body(t_ref, w_ref, o_ref):
    o_ref[0] = _dotT(w_ref[...], t_ref[0])


def _proj_out(t, wout, chunk):
    Bx, S, _ = t.shape
    C = wout.shape[0]
    return pl.pallas_call(
        _proj_body,
        grid=(Bx, S // chunk),
        in_specs=[
            pl.BlockSpec((1, chunk, EMBED), lambda b, s: (b, s, 0)),
            pl.BlockSpec((C, EMBED), lambda b, s: (0, 0)),
        ],
        out_specs=pl.BlockSpec((1, C, chunk), lambda b, s: (b, 0, s)),
        out_shape=jax.ShapeDtypeStruct((Bx, C, S), jnp.float32),
    )(t, wout)


def kernel(feat_p3, feat_p4, feat_p5, Win0, Wout0, Win1, Wout1, Win2, Wout2,
           norm_w, norm_b, ts_W1, ts_b1, ts_W2, ts_b2,
           ca_Wq, ca_bq, ca_Wk, ca_bk, ca_Wv, ca_bv, ca_Wp, ca_bp,
           xa_Wq, xa_bq, xa_Wk, xa_bk, xa_Wv, xa_bv, xa_Wp, xa_bp,
           ffn_W1, ffn_b1, ffn_W2, ffn_b2):
    feats = [feat_p3, feat_p4, feat_p5]
    Wins = [Win0, Win1, Win2]
    Wouts = [Wout0, Wout1, Wout2]
    chunks = [1024, 1024, 256]

    nw = norm_w.reshape(1, EMBED)
    nb = norm_b.reshape(1, EMBED)
    w1 = ts_W1
    b1 = ts_b1.reshape(1, 64)
    w2 = ts_W2.reshape(1, 64)
    b2 = ts_b2.reshape(1, 1)

    ca_ws = (ca_Wq, ca_bq.reshape(1, EMBED), ca_Wk, ca_bk.reshape(1, EMBED),
             ca_Wv, ca_bv.reshape(1, EMBED), ca_Wp, ca_bp.reshape(1, EMBED),
             nw, nb, ffn_W1, ffn_b1.reshape(1, -1), ffn_W2,
             ffn_b2.reshape(1, EMBED))
    xa_ws = (xa_Wq, xa_bq.reshape(1, EMBED), xa_Wk, xa_bk.reshape(1, EMBED),
             xa_Wv, xa_bv.reshape(1, EMBED), xa_Wp, xa_bp.reshape(1, EMBED),
             nw, nb, ffn_W1, ffn_b1.reshape(1, -1), ffn_W2,
             ffn_b2.reshape(1, EMBED))

    tokens, scores = [], []
    for i, x in enumerate(feats):
        Bx, C, Hh, Ww = x.shape
        S = Hh * Ww
        xr = x.reshape(Bx, C, S)
        t, s = _tokenize(xr, Wins[i], nw, nb, w1, b1, w2, b2,
                         min(chunks[i], S))
        tokens.append(t)
        scores.append(s.reshape(Bx, S))

    coarse_idx, fine_idx = [], []
    for i, s in enumerate(scores):
        S = s.shape[1]
        if KF < S:
            ic, fi = _topk_idx(s, (KC, KF))
            fine_idx.append(fi)
        else:
            (ic,) = _topk_idx(s, (KC,))
            fine_idx.append(None)
        coarse_idx.append(ic)

    flat = [t.reshape(-1, EMBED) for t in tokens]
    g0, g1, g2, fs0, fs1 = _gather_all(
        flat[0], flat[1], flat[2],
        coarse_idx[0].reshape(-1), coarse_idx[1].reshape(-1),
        coarse_idx[2].reshape(-1),
        fine_idx[0].reshape(-1), fine_idx[1].reshape(-1))
    cc = jnp.concatenate([g.reshape(B, KC, EMBED) for g in (g0, g1, g2)],
                         axis=1)
    cu = _coarse_attn(cc, ca_ws)

    new0 = _cross_attn(fs0.reshape(B, KF, EMBED), cu, xa_ws)
    new1 = _cross_attn(fs1.reshape(B, KF, EMBED), cu, xa_ws)
    ot0, ot1 = _scatter_rows(
        flat[0], fine_idx[0].reshape(-1), new0.reshape(-1, EMBED),
        tokens[0].shape[1],
        flat[1], fine_idx[1].reshape(-1), new1.reshape(-1, EMBED),
        tokens[1].shape[1])
    out_ts = [ot0.reshape(B, -1, EMBED), ot1.reshape(B, -1, EMBED),
              _cross_attn(tokens[2], cu, xa_ws)]

    outs = []
    for i, out_t in enumerate(out_ts):
        S = out_t.shape[1]
        om = _proj_out(out_t, Wouts[i], min(chunks[i], S))
        Bx, C, Hh, Ww = feats[i].shape
        outs.append(om.reshape(Bx, C, Hh, Ww))
    return tuple(outs)


# speedup vs baseline: 3.5958x; 3.5958x over previous
import functools

import jax
import jax.numpy as jnp
from jax import lax
from jax.experimental import pallas as pl
from jax.experimental.pallas import tpu as pltpu
from jax.experimental.pallas import tpu_sc as plsc

EMBED = 256
HEADS = 8
HD = EMBED // HEADS
KC = 128
KF = 512
B = 8

_PREC = lax.Precision.DEFAULT


def _dot(a, b):
    return lax.dot_general(a, b, (((1,), (0,)), ((), ())),
                           preferred_element_type=jnp.float32, precision=_PREC)


def _dotT(a, b):
    return lax.dot_general(a, b, (((1,), (1,)), ((), ())),
                           preferred_element_type=jnp.float32, precision=_PREC)


def _dotL(a, b):
    return lax.dot_general(a, b, (((0,), (1,)), ((), ())),
                           preferred_element_type=jnp.float32, precision=_PREC)


def _ln_rows(x, w, b):
    m = jnp.mean(x, axis=-1, keepdims=True)
    v = jnp.mean((x - m) ** 2, axis=-1, keepdims=True)
    return (x - m) / jnp.sqrt(v + 1e-5) * w + b


def _silu(x):
    return x * jax.nn.sigmoid(x)


def _tok_body(x_ref, win_ref, nw_ref, nb_ref, w1_ref, b1_ref, w2_ref, b2_ref,
              t_ref, s_ref):
    x = x_ref[0]
    t = _dotL(x, win_ref[...])
    t_ref[0] = t
    m = jnp.sum(t, axis=-1, keepdims=True) * (1.0 / EMBED)
    v = jnp.sum(t * t, axis=-1, keepdims=True) * (1.0 / EMBED) - m * m
    inv = 1.0 / jnp.sqrt(v + 1e-5)
    nt = (t - m) * inv * nw_ref[...] + nb_ref[...]
    h = _silu(_dotT(nt, w1_ref[...]) + b1_ref[...])
    hb = h.astype(jnp.bfloat16).astype(jnp.float32)
    wb = w2_ref[...].astype(jnp.bfloat16).astype(jnp.float32)
    s = jnp.sum(hb * wb, axis=-1) + b2_ref[0, 0]
    s_ref[0, 0, 0] = s


def _tokenize(x, win, nw, nb, w1, b1, w2, b2, chunk):
    Bx, C, S = x.shape
    grid = (Bx, S // chunk)
    return pl.pallas_call(
        _tok_body,
        grid=grid,
        in_specs=[
            pl.BlockSpec((1, C, chunk), lambda b, s: (b, 0, s)),
            pl.BlockSpec((EMBED, C), lambda b, s: (0, 0)),
            pl.BlockSpec((1, EMBED), lambda b, s: (0, 0)),
            pl.BlockSpec((1, EMBED), lambda b, s: (0, 0)),
            pl.BlockSpec((64, EMBED), lambda b, s: (0, 0)),
            pl.BlockSpec((1, 64), lambda b, s: (0, 0)),
            pl.BlockSpec((1, 64), lambda b, s: (0, 0)),
            pl.BlockSpec((1, 1), lambda b, s: (0, 0), memory_space=pltpu.SMEM),
        ],
        out_specs=[
            pl.BlockSpec((1, chunk, EMBED), lambda b, s: (b, s, 0)),
            pl.BlockSpec((1, 1, 1, chunk), lambda b, s: (b, s, 0, 0)),
        ],
        out_shape=[
            jax.ShapeDtypeStruct((Bx, S, EMBED), jnp.float32),
            jax.ShapeDtypeStruct((Bx, S // chunk, 1, chunk), jnp.float32),
        ],
    )(x, win, nw, nb, w1, b1, w2, b2)


def _monotone_u32(s):
    raw = lax.bitcast_convert_type(s, jnp.uint32)
    top = jnp.uint32(0x80000000)
    return jnp.where(raw >= top, ~raw, raw | top)


def _cumsum_lanes(x):
    Bx, S = x.shape
    d = 1
    while d < S:
        sh = jnp.concatenate([jnp.zeros((Bx, d), x.dtype), x[:, :-d]], axis=1)
        x = x + sh
        d *= 2
    return x


def _kth_threshold(u, k):
    Bx = u.shape[0]
    T0 = jnp.zeros((Bx, 1), jnp.uint32)

    def body(i, T):
        bit = lax.shift_left(jnp.uint32(1), jnp.uint32(31) - i.astype(jnp.uint32))
        Ttry = T | bit
        c = jnp.sum((u >= Ttry).astype(jnp.int32), axis=1, keepdims=True)
        return jnp.where(c >= k, Ttry, T)

    return lax.fori_loop(0, 32, body, T0)


def _select_mask(u, k):
    T = _kth_threshold(u, k)
    gt = u > T
    eq = u == T
    n_gt = jnp.sum(gt.astype(jnp.int32), axis=1, keepdims=True)
    need = k - n_gt
    excl = _cumsum_lanes(eq.astype(jnp.int32)) - eq.astype(jnp.int32)
    return gt | (eq & (excl < need))


def _compact(sel, k, s_chunk):
    Bx, S = sel.shape
    c = _cumsum_lanes(sel.astype(jnp.float32))
    jg = lax.broadcasted_iota(jnp.int32, (1, 1, k), 2).astype(jnp.float32)
    acc = jnp.zeros((Bx, k), jnp.float32)
    for s0 in range(0, S, s_chunk):
        cc = c[:, s0:s0 + s_chunk, None]
        acc = acc + jnp.sum((cc <= jg).astype(jnp.float32), axis=1)
    return acc.astype(jnp.int32)


def _select_body(ks, s_ref, *out_refs):
    s = s_ref[...]
    u = _monotone_u32(s)
    Bx, S = s.shape
    for k, ref in zip(ks, out_refs):
        sel = _select_mask(u, k)
        loc = _compact(sel, k, min(S, 512))
        ref[...] = loc + S * lax.broadcasted_iota(jnp.int32, (Bx, k), 0)


def _topk_idx(scores, ks):
    Bx, S = scores.shape
    return pl.pallas_call(
        functools.partial(_select_body, ks),
        in_specs=[pl.BlockSpec((Bx, S), lambda: (0, 0))],
        out_specs=[pl.BlockSpec((Bx, k), lambda: (0, 0)) for k in ks],
        out_shape=[jax.ShapeDtypeStruct((Bx, k), jnp.int32) for k in ks],
    )(scores)


_SC_MESH = plsc.VectorSubcoreMesh(core_axis_name="c", subcore_axis_name="s")


def _gather_all(t0, t1, t2, ic0, ic1, ic2, fi0, fi1):
    CPER = B * KC // 32
    FPER = B * KF // 32

    @functools.partial(
        pl.kernel,
        out_type=(jax.ShapeDtypeStruct((B * KC, EMBED), jnp.float32),
                  jax.ShapeDtypeStruct((B * KC, EMBED), jnp.float32),
                  jax.ShapeDtypeStruct((B * KC, EMBED), jnp.float32),
                  jax.ShapeDtypeStruct((B * KF, EMBED), jnp.float32),
                  jax.ShapeDtypeStruct((B * KF, EMBED), jnp.float32)),
        mesh=_SC_MESH,
        scratch_types=[
            pltpu.VMEM((CPER,), jnp.int32),
            pltpu.VMEM((CPER, EMBED), jnp.float32),
            pltpu.VMEM((FPER,), jnp.int32),
            pltpu.VMEM((FPER, EMBED), jnp.float32),
            pltpu.SemaphoreType.DMA,
        ],
    )
    def k(t0_h, t1_h, t2_h, i0_h, i1_h, i2_h, f0_h, f1_h,
          g0_h, g1_h, g2_h, o0_h, o1_h, ci_v, cr_v, fi_v, fr_v, sem):
        w = lax.axis_index("s") * 2 + lax.axis_index("c")
        cbase = w * CPER
        fbase = w * FPER
        for th, ih, gh in ((t0_h, i0_h, g0_h), (t1_h, i1_h, g1_h),
                           (t2_h, i2_h, g2_h)):
            pltpu.sync_copy(ih.at[pl.ds(cbase, CPER)], ci_v)
            pltpu.async_copy(th.at[ci_v], cr_v, sem).wait()
            pltpu.sync_copy(cr_v, gh.at[pl.ds(cbase, CPER)])
        for th, fh, oh in ((t0_h, f0_h, o0_h), (t1_h, f1_h, o1_h)):
            pltpu.sync_copy(fh.at[pl.ds(fbase, FPER)], fi_v)
            pltpu.async_copy(th.at[fi_v], fr_v, sem).wait()
            pltpu.sync_copy(fr_v, oh.at[pl.ds(fbase, FPER)])

    return k(t0, t1, t2, ic0, ic1, ic2, fi0, fi1)


def _scatter_rows(t0, fi0, rows0, S0, t1, fi1, rows1, S1):
    SCAT = 4 * KF // 16

    @functools.partial(
        pl.kernel,
        out_type=(jax.ShapeDtypeStruct((B * S0, EMBED), jnp.float32),
                  jax.ShapeDtypeStruct((B * S1, EMBED), jnp.float32)),
        mesh=_SC_MESH,
        scratch_types=[
            pltpu.VMEM((SCAT,), jnp.int32),
            pltpu.VMEM((SCAT, EMBED), jnp.float32),
            pltpu.VMEM((128, EMBED), jnp.float32),
            pltpu.SemaphoreType.DMA,
        ],
    )
    def k(t0_h, f0_h, r0_h, t1_h, f1_h, r1_h, o0_h, o1_h,
          idx_v, rbuf_v, buf_v, sem):
        cid = lax.axis_index("c")
        sid = lax.axis_index("s")
        for S, t_h, o_h in ((S0, t0_h, o0_h), (S1, t1_h, o1_h)):
            cp = S // 4
            cbase = cid * (4 * S) + sid * cp
            for j in range(cp // 128):
                pltpu.sync_copy(t_h.at[pl.ds(cbase + j * 128, 128)], buf_v)
                pltpu.sync_copy(buf_v, o_h.at[pl.ds(cbase + j * 128, 128)])
        plsc.subcore_barrier()
        sbase = cid * (4 * KF) + sid * SCAT
        for f_h, r_h, o_h in ((f0_h, r0_h, o0_h), (f1_h, r1_h, o1_h)):
            pltpu.sync_copy(f_h.at[pl.ds(sbase, SCAT)], idx_v)
            pltpu.sync_copy(r_h.at[pl.ds(sbase, SCAT)], rbuf_v)
            pltpu.async_copy(rbuf_v, o_h.at[idx_v], sem).wait()

    return k(t0, fi0, rows0, t1, fi1, rows1)


def _attn(q, k, v):
    o = []
    sc = HD ** (-0.5)
    for h in range(HEADS):
        qh = q[:, h * HD:(h + 1) * HD]
        kh = k[:, h * HD:(h + 1) * HD]
        vh = v[:, h * HD:(h + 1) * HD]
        e = jnp.exp(_dotT(qh, kh) * sc)
        r = 1.0 / jnp.sum(e, axis=-1, keepdims=True)
        o.append(_dot(e, vh) * r)
    return jnp.concatenate(o, axis=-1)


def _coarse_body(cc_ref, wq, bq, wk, bk, wv, bv, wp, bp, nw, nb,
                 fw1, fb1, fw2, fb2, cu_ref):
    X = cc_ref[0]
    q = _dotT(X, wq[...]) + bq[...]
    k = _dotT(X, wk[...]) + bk[...]
    v = _dotT(X, wv[...]) + bv[...]
    o = _attn(q, k, v)
    cu0 = _dotT(o, wp[...]) + bp[...]
    f = _silu(_dotT(_ln_rows(cu0, nw[...], nb[...]), fw1[...]) + fb1[...])
    cu_ref[0] = _dotT(f, fw2[...]) + fb2[...] + cu0


def _coarse_attn(cc, ws):
    N = cc.shape[1]
    specs = [pl.BlockSpec((1, N, EMBED), lambda b: (b, 0, 0))]
    for w in ws:
        sh = w.shape
        specs.append(pl.BlockSpec(sh, lambda b: (0,) * len(sh)))
    return pl.pallas_call(
        _coarse_body,
        grid=(cc.shape[0],),
        in_specs=specs,
        out_specs=pl.BlockSpec((1, N, EMBED), lambda b: (b, 0, 0)),
        out_shape=jax.ShapeDtypeStruct(cc.shape, jnp.float32),
    )(cc, *ws)


def _cross_body(fs_ref, cu_ref, wq, bq, wk, bk, wv, bv, wp, bp, nw, nb,
                fw1, fb1, fw2, fb2, out_ref):
    F = fs_ref[0]
    CU = cu_ref[0]
    q = _dotT(F, wq[...]) + bq[...]
    k = _dotT(CU, wk[...]) + bk[...]
    v = _dotT(CU, wv[...]) + bv[...]
    o = _attn(q, k, v)
    up0 = _dotT(o, wp[...]) + bp[...]
    f = _silu(_dotT(_ln_rows(up0, nw[...], nb[...]), fw1[...]) + fb1[...])
    up = _dotT(f, fw2[...]) + fb2[...] + up0
    out_ref[0] = F + up


def _cross_attn(fs, cu, ws):
    N = fs.shape[1]
    M = cu.shape[1]
    specs = [pl.BlockSpec((1, N, EMBED), lambda b: (b, 0, 0)),
             pl.BlockSpec((1, M, EMBED), lambda b: (b, 0, 0))]
    for w in ws:
        sh = w.shape
        specs.append(pl.BlockSpec(sh, lambda b: (0,) * len(sh)))
    return pl.pallas_call(
        _cross_body,
        grid=(fs.shape[0],),
        in_specs=specs,
        out_specs=pl.BlockSpec((1, N, EMBED), lambda b: (b, 0, 0)),
        out_shape=jax.ShapeDtypeStruct(fs.shape, jnp.float32),
    )(fs, cu, *ws)


def _proj_body(t_ref, w_ref, o_ref):
    o_ref[0] = _dotT(w_ref[...], t_ref[0])


def _proj_out(t, wout, chunk):
    Bx, S, _ = t.shape
    C = wout.shape[0]
    return pl.pallas_call(
        _proj_body,
        grid=(Bx, S // chunk),
        in_specs=[
            pl.BlockSpec((1, chunk, EMBED), lambda b, s: (b, s, 0)),
            pl.BlockSpec((C, EMBED), lambda b, s: (0, 0)),
        ],
        out_specs=pl.BlockSpec((1, C, chunk), lambda b, s: (b, 0, s)),
        out_shape=jax.ShapeDtypeStruct((Bx, C, S), jnp.float32),
    )(t, wout)


def kernel(feat_p3, feat_p4, feat_p5, Win0, Wout0, Win1, Wout1, Win2, Wout2,
           norm_w, norm_b, ts_W1, ts_b1, ts_W2, ts_b2,
           ca_Wq, ca_bq, ca_Wk, ca_bk, ca_Wv, ca_bv, ca_Wp, ca_bp,
           xa_Wq, xa_bq, xa_Wk, xa_bk, xa_Wv, xa_bv, xa_Wp, xa_bp,
           ffn_W1, ffn_b1, ffn_W2, ffn_b2):
    feats = [feat_p3, feat_p4, feat_p5]
    Wins = [Win0, Win1, Win2]
    Wouts = [Wout0, Wout1, Wout2]
    chunks = [1024, 1024, 256]

    nw = norm_w.reshape(1, EMBED)
    nb = norm_b.reshape(1, EMBED)
    w1 = ts_W1
    b1 = ts_b1.reshape(1, 64)
    w2 = ts_W2.reshape(1, 64)
    b2 = ts_b2.reshape(1, 1)

    ca_ws = (ca_Wq, ca_bq.reshape(1, EMBED), ca_Wk, ca_bk.reshape(1, EMBED),
             ca_Wv, ca_bv.reshape(1, EMBED), ca_Wp, ca_bp.reshape(1, EMBED),
             nw, nb, ffn_W1, ffn_b1.reshape(1, -1), ffn_W2,
             ffn_b2.reshape(1, EMBED))
    xa_ws = (xa_Wq, xa_bq.reshape(1, EMBED), xa_Wk, xa_bk.reshape(1, EMBED),
             xa_Wv, xa_bv.reshape(1, EMBED), xa_Wp, xa_bp.reshape(1, EMBED),
             nw, nb, ffn_W1, ffn_b1.reshape(1, -1), ffn_W2,
             ffn_b2.reshape(1, EMBED))

    tokens, scores = [], []
    for i, x in enumerate(feats):
        Bx, C, Hh, Ww = x.shape
        S = Hh * Ww
        xr = x.reshape(Bx, C, S)
        t, s = _tokenize(xr, Wins[i], nw, nb, w1, b1, w2, b2,
                         min(chunks[i], S))
        tokens.append(t)
        scores.append(s.reshape(Bx, S))

    coarse_idx, fine_idx = [], []
    for i, s in enumerate(scores):
        S = s.shape[1]
        if KF < S:
            ic, fi = _topk_idx(s, (KC, KF))
            fine_idx.append(fi)
        else:
            (ic,) = _topk_idx(s, (KC,))
            fine_idx.append(None)
        coarse_idx.append(ic)

    flat = [t.reshape(-1, EMBED) for t in tokens]
    g0, g1, g2, fs0, fs1 = _gather_all(
        flat[0], flat[1], flat[2],
        coarse_idx[0].reshape(-1), coarse_idx[1].reshape(-1),
        coarse_idx[2].reshape(-1),
        fine_idx[0].reshape(-1), fine_idx[1].reshape(-1))
    cc = jnp.concatenate([g.reshape(B, KC, EMBED) for g in (g0, g1, g2)],
                         axis=1)
    cu = _coarse_attn(cc, ca_ws)

    new0 = _cross_attn(fs0.reshape(B, KF, EMBED), cu, xa_ws)
    new1 = _cross_attn(fs1.reshape(B, KF, EMBED), cu, xa_ws)
    ot0, ot1 = _scatter_rows(
        flat[0], fine_idx[0].reshape(-1), new0.reshape(-1, EMBED),
        tokens[0].shape[1],
        flat[1], fine_idx[1].reshape(-1), new1.reshape(-1, EMBED),
        tokens[1].shape[1])
    out_ts = [ot0.reshape(B, -1, EMBED), ot1.reshape(B, -1, EMBED),
              _cross_attn(tokens[2], cu, xa_ws)]

    outs = []
    for i, out_t in enumerate(out_ts):
        S = out_t.shape[1]
        om = _proj_out(out_t, Wouts[i], min(chunks[i], S))
        Bx, C, Hh, Ww = feats[i].shape
        outs.append(om.reshape(Bx, C, Hh, Ww))
    return tuple(outs)
